# baseline (speedup 1.0000x reference)
import numpy as np
import jax
import jax.numpy as jnp
from jax.experimental import pallas as pl
from jax.experimental.pallas import tpu as pltpu

N = 16384
T = 1000
S = 1000
H = 64
B = 256
STEPS = N // B
SMOOTH = np.float32(0.1)
KEEP = np.float32(1.0 - 0.1)
TINY = np.float32(np.finfo(np.float32).tiny)
DELTA = np.float32(np.float32(1.0) - TINY)


def _np_threefry2x32(k0, k1, x0, x1):
    def rotl(x, d):
        return ((x << np.uint32(d)) | (x >> np.uint32(32 - d))).astype(np.uint32)
    ks = [np.uint32(k0), np.uint32(k1),
          np.uint32(np.uint32(k0) ^ np.uint32(k1) ^ np.uint32(0x1BD11BDA))]
    x0 = (np.uint32(x0) + ks[0]).astype(np.uint32)
    x1 = (np.uint32(x1) + ks[1]).astype(np.uint32)
    rotations = [[13, 15, 26, 6], [17, 29, 16, 24]]
    for i in range(5):
        for r in rotations[i % 2]:
            x0 = (x0 + x1).astype(np.uint32)
            x1 = rotl(x1, r)
            x1 = x0 ^ x1
        x0 = (x0 + ks[(i + 1) % 3]).astype(np.uint32)
        x1 = (x1 + ks[(i + 2) % 3] + np.uint32(i + 1)).astype(np.uint32)
    return x0, x1


_SEED_KEY = (np.uint32(0), np.uint32(42))
_KO = _np_threefry2x32(_SEED_KEY[0], _SEED_KEY[1], np.uint32(0), np.uint32(0))
_KS = _np_threefry2x32(_SEED_KEY[0], _SEED_KEY[1], np.uint32(0), np.uint32(1))
K_OP0, K_OP1 = int(_KO[0]), int(_KO[1])
K_SC0, K_SC1 = int(_KS[0]), int(_KS[1])


def _tf_bits(k0, k1, flat):
    ks0 = jnp.uint32(k0)
    ks1 = jnp.uint32(k1)
    ks2 = jnp.uint32(k0 ^ k1 ^ 0x1BD11BDA)
    ks = (ks0, ks1, ks2)
    x0 = jnp.full_like(flat, ks0)
    x1 = flat + ks1
    rotations = ((13, 15, 26, 6), (17, 29, 16, 24))
    for i in range(5):
        for r in rotations[i % 2]:
            x0 = x0 + x1
            x1 = (x1 << r) | (x1 >> (32 - r))
            x1 = x0 ^ x1
        x0 = x0 + ks[(i + 1) % 3]
        x1 = x1 + ks[(i + 2) % 3] + jnp.uint32(i + 1)
    return x0 ^ x1


def _log_u_from_bits(bits):
    fb = (bits >> jnp.uint32(9)) | jnp.uint32(0x3F800000)
    f = jax.lax.bitcast_convert_type(fb, jnp.float32) - jnp.float32(1.0)
    return jnp.log(f + TINY)


def _first_argmax(a, col):
    m = jnp.max(a, axis=1, keepdims=True)
    return jnp.min(jnp.where(a == m, col, jnp.int32(a.shape[1])), axis=1)


def _sampler_kernel(op_embs_ref, scale_embs_ref, q_ref,
                    op_idx_ref, sc_idx_ref, logps_ref,
                    lp_op_ref, flat_ref, acc_ref):
    i = pl.program_id(0)

    @pl.when(i == 0)
    def _prologue():
        q2 = q_ref[...]
        opl = jax.lax.dot_general(
            q2, op_embs_ref[...], (((1,), (1,)), ((), ())),
            preferred_element_type=jnp.float32,
            precision=jax.lax.Precision.DEFAULT)
        x_max = jnp.max(opl, axis=1, keepdims=True)
        shifted = opl - x_max
        lse = jnp.log(jnp.sum(jnp.exp(shifted), axis=1, keepdims=True))
        lp_op_ref[...] = shifted - lse
        acc_ref[0, 0] = jnp.float32(0.0)

    col = jax.lax.broadcasted_iota(jnp.int32, (B, T), 1)

    @pl.when(i == 0)
    def _init_flat():
        row = jax.lax.broadcasted_iota(jnp.int32, (B, T), 0)
        flat_ref[...] = (row * T + col).astype(jnp.uint32)

    flat = flat_ref[...]
    flat_ref[...] = flat + jnp.uint32(B * T)

    s1 = lp_op_ref[...] - jnp.log(-_log_u_from_bits(_tf_bits(K_OP0, K_OP1, flat)))
    op_idx = _first_argmax(s1, col)

    onehot = (col == op_idx[:, None]).astype(jnp.float32)
    hidden = jax.lax.dot_general(
        onehot, op_embs_ref[...], (((1,), (0,)), ((), ())),
        preferred_element_type=jnp.float32,
        precision=jax.lax.Precision.HIGHEST) + q_ref[...]
    lp_op_at = jnp.sum(onehot * lp_op_ref[...], axis=1, keepdims=True)

    sl = jax.lax.dot_general(
        hidden, scale_embs_ref[...], (((1,), (1,)), ((), ())),
        preferred_element_type=jnp.float32,
        precision=jax.lax.Precision.DEFAULT)
    m2 = jnp.max(sl, axis=1, keepdims=True)
    sh = sl - m2
    lse2 = jnp.log(jnp.sum(jnp.exp(sh), axis=1, keepdims=True))

    s2 = (sh - lse2) - jnp.log(-_log_u_from_bits(_tf_bits(K_SC0, K_SC1, flat)))
    sc_idx = _first_argmax(s2, col)

    sh_at = jnp.sum(jnp.where(col == sc_idx[:, None], sh, 0.0), axis=1,
                    keepdims=True)
    raw = lp_op_at + (sh_at - lse2)

    acc_ref[0, 0] += jnp.sum(
        (jnp.sum(sh, axis=1, keepdims=True) - jnp.float32(S) * lse2)
        / jnp.float32(S))

    op_idx_ref[pl.ds(i, 1), :] = op_idx[None, :]
    sc_idx_ref[pl.ds(i, 1), :] = sc_idx[None, :]
    logps_ref[pl.ds(i, 1), :] = raw[:, 0][None, :]

    @pl.when(i == STEPS - 1)
    def _epilogue():
        mean_op = jnp.sum(lp_op_ref[...]) / jnp.float32(T)
        big = (mean_op * jnp.float32(N) + acc_ref[0, 0]) * SMOOTH
        logps_ref[...] = big + logps_ref[...] * KEEP


def kernel(op_embs, scale_embs, q, num_samples):
    q2 = q.reshape(1, H)
    full = lambda s: pl.BlockSpec(s, lambda i: (0,) * len(s))
    op_idx, sc_idx, logps = pl.pallas_call(
        _sampler_kernel,
        grid=(STEPS,),
        in_specs=[full((T, H)), full((S, H)), full((1, H))],
        out_specs=[full((STEPS, B)), full((STEPS, B)), full((STEPS, B))],
        out_shape=[
            jax.ShapeDtypeStruct((STEPS, B), jnp.int32),
            jax.ShapeDtypeStruct((STEPS, B), jnp.int32),
            jax.ShapeDtypeStruct((STEPS, B), jnp.float32),
        ],
        scratch_shapes=[
            pltpu.VMEM((1, T), jnp.float32),
            pltpu.VMEM((B, T), jnp.uint32),
            pltpu.SMEM((1, 1), jnp.float32),
        ],
    )(op_embs, scale_embs, q2)
    return op_idx.reshape(N), sc_idx.reshape(N), logps.reshape(N)


# speedup vs baseline: 1.0280x; 1.0280x over previous
import numpy as np
import jax
import jax.numpy as jnp
from jax.experimental import pallas as pl
from jax.experimental.pallas import tpu as pltpu

N = 16384
T = 1000
S = 1000
H = 64
B = 512
STEPS = N // B
SMOOTH = np.float32(0.1)
KEEP = np.float32(1.0 - 0.1)
TINY = np.float32(np.finfo(np.float32).tiny)


def _np_threefry2x32(k0, k1, x0, x1):
    def rotl(x, d):
        return ((x << np.uint32(d)) | (x >> np.uint32(32 - d))).astype(np.uint32)
    ks = [np.uint32(k0), np.uint32(k1),
          np.uint32(np.uint32(k0) ^ np.uint32(k1) ^ np.uint32(0x1BD11BDA))]
    x0 = (np.uint32(x0) + ks[0]).astype(np.uint32)
    x1 = (np.uint32(x1) + ks[1]).astype(np.uint32)
    rotations = [[13, 15, 26, 6], [17, 29, 16, 24]]
    for i in range(5):
        for r in rotations[i % 2]:
            x0 = (x0 + x1).astype(np.uint32)
            x1 = rotl(x1, r)
            x1 = x0 ^ x1
        x0 = (x0 + ks[(i + 1) % 3]).astype(np.uint32)
        x1 = (x1 + ks[(i + 2) % 3] + np.uint32(i + 1)).astype(np.uint32)
    return x0, x1


_SEED_KEY = (np.uint32(0), np.uint32(42))
_KO = _np_threefry2x32(_SEED_KEY[0], _SEED_KEY[1], np.uint32(0), np.uint32(0))
_KS = _np_threefry2x32(_SEED_KEY[0], _SEED_KEY[1], np.uint32(0), np.uint32(1))
K_OP0, K_OP1 = int(_KO[0]), int(_KO[1])
K_SC0, K_SC1 = int(_KS[0]), int(_KS[1])


def _tf_bits(k0, k1, flat):
    ks0 = jnp.uint32(k0)
    ks1 = jnp.uint32(k1)
    ks2 = jnp.uint32(k0 ^ k1 ^ 0x1BD11BDA)
    ks = (ks0, ks1, ks2)
    x0 = jnp.full_like(flat, ks0)
    x1 = flat + ks1
    rotations = ((13, 15, 26, 6), (17, 29, 16, 24))
    for i in range(5):
        for r in rotations[i % 2]:
            x0 = x0 + x1
            x1 = (x1 << r) | (x1 >> (32 - r))
            x1 = x0 ^ x1
        x0 = x0 + ks[(i + 1) % 3]
        x1 = x1 + ks[(i + 2) % 3] + jnp.uint32(i + 1)
    return x0 ^ x1


def _log_u_from_bits(bits):
    fb = (bits >> jnp.uint32(9)) | jnp.uint32(0x3F800000)
    f = jax.lax.bitcast_convert_type(fb, jnp.float32) - jnp.float32(1.0)
    return jnp.log(f + TINY)


def _first_argmax(a, col):
    m = jnp.max(a, axis=1, keepdims=True)
    return jnp.min(jnp.where(a == m, col, jnp.int32(a.shape[1])), axis=1)


def _sampler_kernel(op_embs_ref, scale_embs_ref, q_ref,
                    op_idx_ref, sc_idx_ref, logps_ref,
                    lp_op_ref, flat_ref, acc_ref):
    i = pl.program_id(0)

    @pl.when(i == 0)
    def _prologue():
        q2 = q_ref[...]
        opl = jax.lax.dot_general(
            q2, op_embs_ref[...], (((1,), (1,)), ((), ())),
            preferred_element_type=jnp.float32,
            precision=jax.lax.Precision.DEFAULT)
        x_max = jnp.max(opl, axis=1, keepdims=True)
        shifted = opl - x_max
        lse = jnp.log(jnp.sum(jnp.exp(shifted), axis=1, keepdims=True))
        lp_op_ref[...] = shifted - lse
        acc_ref[0, 0] = jnp.float32(0.0)

    col = jax.lax.broadcasted_iota(jnp.int32, (B, T), 1)

    @pl.when(i == 0)
    def _init_flat():
        row = jax.lax.broadcasted_iota(jnp.int32, (B, T), 0)
        flat_ref[...] = (row * T + col).astype(jnp.uint32)

    flat = flat_ref[...]
    flat_ref[...] = flat + jnp.uint32(B * T)

    s1 = lp_op_ref[...] - jnp.log(-_log_u_from_bits(_tf_bits(K_OP0, K_OP1, flat)))
    op_idx = _first_argmax(s1, col)

    onehot = (col == op_idx[:, None]).astype(jnp.float32)
    hidden = jax.lax.dot_general(
        onehot, op_embs_ref[...], (((1,), (0,)), ((), ())),
        preferred_element_type=jnp.float32,
        precision=jax.lax.Precision.HIGHEST) + q_ref[...]
    lp_op_at = jnp.sum(onehot * lp_op_ref[...], axis=1, keepdims=True)

    sl = jax.lax.dot_general(
        hidden, scale_embs_ref[...], (((1,), (1,)), ((), ())),
        preferred_element_type=jnp.float32,
        precision=jax.lax.Precision.DEFAULT)
    m2 = jnp.max(sl, axis=1, keepdims=True)
    sh = sl - m2
    lse2 = jnp.log(jnp.sum(jnp.exp(sh), axis=1, keepdims=True))

    s2 = (sh - lse2) - jnp.log(-_log_u_from_bits(_tf_bits(K_SC0, K_SC1, flat)))
    sc_idx = _first_argmax(s2, col)

    sh_at = jnp.sum(jnp.where(col == sc_idx[:, None], sh, 0.0), axis=1,
                    keepdims=True)
    raw = lp_op_at + (sh_at - lse2)

    acc_ref[0, 0] += jnp.sum(
        (jnp.sum(sh, axis=1, keepdims=True) - jnp.float32(S) * lse2)
        / jnp.float32(S))

    op_idx_ref[pl.ds(i, 1), :] = op_idx[None, :]
    sc_idx_ref[pl.ds(i, 1), :] = sc_idx[None, :]
    logps_ref[pl.ds(i, 1), :] = raw[:, 0][None, :]

    @pl.when(i == STEPS - 1)
    def _epilogue():
        mean_op = jnp.sum(lp_op_ref[...]) / jnp.float32(T)
        big = (mean_op * jnp.float32(N) + acc_ref[0, 0]) * SMOOTH
        logps_ref[...] = big + logps_ref[...] * KEEP


def kernel(op_embs, scale_embs, q, num_samples):
    q2 = q.reshape(1, H)
    full = lambda s: pl.BlockSpec(s, lambda i: (0,) * len(s))
    op_idx, sc_idx, logps = pl.pallas_call(
        _sampler_kernel,
        grid=(STEPS,),
        in_specs=[full((T, H)), full((S, H)), full((1, H))],
        out_specs=[full((STEPS, B)), full((STEPS, B)), full((STEPS, B))],
        out_shape=[
            jax.ShapeDtypeStruct((STEPS, B), jnp.int32),
            jax.ShapeDtypeStruct((STEPS, B), jnp.int32),
            jax.ShapeDtypeStruct((STEPS, B), jnp.float32),
        ],
        scratch_shapes=[
            pltpu.VMEM((1, T), jnp.float32),
            pltpu.VMEM((B, T), jnp.uint32),
            pltpu.SMEM((1, 1), jnp.float32),
        ],
    )(op_embs, scale_embs, q2)
    return op_idx.reshape(N), sc_idx.reshape(N), logps.reshape(N)
